# speedup vs baseline: 2.9682x; 2.9682x over previous
import jax
import jax.numpy as jnp
from jax.experimental import pallas as pl
from jax.experimental.pallas import tpu as pltpu

_N = 8192
_D = 256
_K = 512
_MAX_ITER = 10
_ROWS = 512


def _kmeans_body(x_ref, cinit_ref, out_ref, centers, sums, counts):
    it = pl.program_id(0)

    @pl.when(it == 0)
    def _init():
        centers[...] = cinit_ref[...]

    sums[...] = jnp.zeros_like(sums)
    counts[...] = jnp.zeros_like(counts)

    c = centers[...]
    c2 = jnp.sum(c * c, axis=1)[None, :]

    def body(i, carry):
        xb = x_ref[pl.ds(i * _ROWS, _ROWS), :]
        x2 = jnp.sum(xb * xb, axis=1, keepdims=True)
        xc = jax.lax.dot_general(
            xb, c, (((1,), (1,)), ((), ())),
            precision=jax.lax.Precision.DEFAULT,
            preferred_element_type=jnp.float32,
        )
        d2 = x2 + c2 - 2.0 * xc
        m = jnp.min(d2, axis=1, keepdims=True)
        iota = jax.lax.broadcasted_iota(jnp.int32, (_ROWS, _K), 1)
        masked = jnp.where(d2 <= m, iota, _K)
        assign = jnp.min(masked, axis=1, keepdims=True)
        onehot = (iota == assign).astype(jnp.float32)
        sums[...] += jax.lax.dot_general(
            onehot, xb, (((0,), (0,)), ((), ())),
            precision=jax.lax.Precision.HIGHEST,
            preferred_element_type=jnp.float32,
        )
        counts[...] += jax.lax.dot_general(
            onehot, jnp.ones((_ROWS, 8), jnp.float32),
            (((0,), (0,)), ((), ())),
            precision=jax.lax.Precision.HIGHEST,
            preferred_element_type=jnp.float32,
        )
        return carry

    jax.lax.fori_loop(0, _N // _ROWS, body, 0)

    cnt = jnp.maximum(counts[:, 0:1], 1.0)
    newc = sums[...] / cnt
    centers[...] = newc
    out_ref[...] = newc


@jax.jit
def kernel(x):
    perm = jax.random.permutation(jax.random.key(1), x.shape[0])[:_K]
    cinit = x[perm]
    return pl.pallas_call(
        _kmeans_body,
        grid=(_MAX_ITER,),
        in_specs=[
            pl.BlockSpec((_N, _D), lambda i: (0, 0)),
            pl.BlockSpec((_K, _D), lambda i: (0, 0)),
        ],
        out_specs=pl.BlockSpec((_K, _D), lambda i: (0, 0)),
        out_shape=jax.ShapeDtypeStruct((_K, _D), jnp.float32),
        scratch_shapes=[
            pltpu.VMEM((_K, _D), jnp.float32),
            pltpu.VMEM((_K, _D), jnp.float32),
            pltpu.VMEM((_K, 8), jnp.float32),
        ],
    )(x, cinit)


# speedup vs baseline: 5.9860x; 2.0167x over previous
import jax
import jax.numpy as jnp
from jax.experimental import pallas as pl
from jax.experimental.pallas import tpu as pltpu

_N = 8192
_D = 256
_K = 512
_MAX_ITER = 10
_ROWS = 512


def _kmeans_body(x_ref, cinit_ref, out_ref, centers, sums, counts):
    it = pl.program_id(0)

    @pl.when(it == 0)
    def _init():
        centers[...] = cinit_ref[...]

    sums[...] = jnp.zeros_like(sums)
    counts[...] = jnp.zeros_like(counts)

    c = centers[...]
    c2 = jnp.sum(c * c, axis=1)[None, :]
    iota = jax.lax.broadcasted_iota(jnp.int32, (_ROWS, _K), 1)
    ones8 = jnp.ones((_ROWS, 8), jnp.float32)

    def body(i, carry):
        xb = x_ref[pl.ds(i * _ROWS, _ROWS), :]
        x2 = jnp.sum(xb * xb, axis=1, keepdims=True)
        xc = jax.lax.dot_general(
            xb, c, (((1,), (1,)), ((), ())),
            precision=jax.lax.Precision.DEFAULT,
            preferred_element_type=jnp.float32,
        )
        d2 = x2 + c2 - 2.0 * xc
        m = jnp.min(d2, axis=1, keepdims=True)
        masked = jnp.where(d2 <= m, iota, _K)
        assign = jnp.min(masked, axis=1, keepdims=True)
        onehot = jnp.where(iota == assign, 1.0, 0.0)
        sums[...] += jax.lax.dot_general(
            onehot, xb, (((0,), (0,)), ((), ())),
            precision=jax.lax.Precision.DEFAULT,
            preferred_element_type=jnp.float32,
        )
        counts[...] += jax.lax.dot_general(
            onehot, ones8, (((0,), (0,)), ((), ())),
            precision=jax.lax.Precision.DEFAULT,
            preferred_element_type=jnp.float32,
        )
        return carry

    jax.lax.fori_loop(0, _N // _ROWS, body, 0)

    cnt = jnp.maximum(counts[:, 0:1], 1.0)
    newc = sums[...] / cnt
    centers[...] = newc
    out_ref[...] = newc


@jax.jit
def kernel(x):
    perm = jax.random.permutation(jax.random.key(1), x.shape[0])[:_K]
    cinit = x[perm]
    return pl.pallas_call(
        _kmeans_body,
        grid=(_MAX_ITER,),
        in_specs=[
            pl.BlockSpec((_N, _D), lambda i: (0, 0)),
            pl.BlockSpec((_K, _D), lambda i: (0, 0)),
        ],
        out_specs=pl.BlockSpec((_K, _D), lambda i: (0, 0)),
        out_shape=jax.ShapeDtypeStruct((_K, _D), jnp.float32),
        scratch_shapes=[
            pltpu.VMEM((_K, _D), jnp.float32),
            pltpu.VMEM((_K, _D), jnp.float32),
            pltpu.VMEM((_K, 8), jnp.float32),
        ],
    )(x, cinit)
